# Initial kernel scaffold; baseline (speedup 1.0000x reference)
import jax
import jax.numpy as jnp
from jax.experimental import pallas as pl


def kernel(indices, tables):
    raise NotImplementedError("write your pallas kernel here")


# speedup vs baseline: 1.6020x; 1.6020x over previous
import functools

import jax
import jax.numpy as jnp
from jax import lax
from jax.experimental import pallas as pl
from jax.experimental.pallas import tpu as pltpu
from jax.experimental.pallas import tpu_sc as plsc

NUM_TABLES = 26
VOCAB = 100000
DIM = 64
BATCH = 4096
L = 20

NC = 2
NS = 16
NW = NC * NS
LANES = 16
ROWP = 2 * DIM

BAGS_PER_W = BATCH // NW
CHUNK = 8
BLOCKS = BAGS_PER_W // CHUNK
PAIRS = NUM_TABLES // 2
N_CHUNKS = PAIRS * BLOCKS
ROWS_PER_CHUNK = 2 * CHUNK * L
IDX_ROWS = 3
GSIZES = (128, 128, 64)
TOTAL_CHUNKS = NW * N_CHUNKS


def _emb_body(idx_hbm, tbl_hbm, out_hbm,
              idx0, idx1, rows0, rows1, ob0, ob1,
              isem0, isem1, gsem0, gsem1, osem0, osem1):
  w = lax.axis_index("s") * NC + lax.axis_index("c")

  def idx_cp(i, ib, sem):
    return pltpu.make_async_copy(idx_hbm.at[w * N_CHUNKS + i], ib, sem)

  def gath(ib, rb, sem, j):
    sz = GSIZES[j]
    return pltpu.make_async_copy(
        tbl_hbm.at[ib.at[j, pl.ds(0, sz)]], rb.at[pl.ds(j * 128, sz)], sem)

  def out_cp(i, ob, sem):
    p = i // BLOCKS
    c = i % BLOCKS
    b0 = w * BAGS_PER_W + c * CHUNK
    return pltpu.make_async_copy(
        ob, out_hbm.at[pl.ds(b0, CHUNK), pl.ds(p * ROWP, ROWP)], sem)

  def accumulate(rb, ob):
    def bag(c, carry):
      for h in range(2):
        base = h * (CHUNK * L) + c * L
        for d in range(DIM // LANES):
          acc = rb[base, pl.ds(d * LANES, LANES)]
          for l in range(1, L):
            acc = acc + rb[base + l, pl.ds(d * LANES, LANES)]
          ob[c, pl.ds(h * DIM + d * LANES, LANES)] = acc
      return carry
    lax.fori_loop(0, CHUNK, bag, 0)

  idx_cp(0, idx0, isem0).start()
  idx_cp(0, idx0, isem0).wait()
  for j in range(len(GSIZES)):
    gath(idx0, rows0, gsem0, j).start()
  idx_cp(1, idx1, isem1).start()

  def step(i2, carry):
    i = i2 * 2

    idx_cp(i + 1, idx1, isem1).wait()
    for j in range(len(GSIZES)):
      gath(idx1, rows1, gsem1, j).start()
    for j in range(len(GSIZES)):
      gath(idx0, rows0, gsem0, j).wait()

    @pl.when(i + 2 < N_CHUNKS)
    def _():
      idx_cp(i + 2, idx0, isem0).start()

    @pl.when(i >= 2)
    def _():
      out_cp(i - 2, ob0, osem0).wait()

    accumulate(rows0, ob0)
    out_cp(i, ob0, osem0).start()

    @pl.when(i + 2 < N_CHUNKS)
    def _():
      idx_cp(i + 2, idx0, isem0).wait()
      for j in range(len(GSIZES)):
        gath(idx0, rows0, gsem0, j).start()

    for j in range(len(GSIZES)):
      gath(idx1, rows1, gsem1, j).wait()

    @pl.when(i + 3 < N_CHUNKS)
    def _():
      idx_cp(i + 3, idx1, isem1).start()

    @pl.when(i >= 2)
    def _():
      out_cp(i - 1, ob1, osem1).wait()

    accumulate(rows1, ob1)
    out_cp(i + 1, ob1, osem1).start()
    return carry

  lax.fori_loop(0, N_CHUNKS // 2, step, 0)

  out_cp(N_CHUNKS - 2, ob0, osem0).wait()
  out_cp(N_CHUNKS - 1, ob1, osem1).wait()


_emb_kernel = pl.kernel(
    _emb_body,
    out_type=jax.ShapeDtypeStruct((BATCH, NUM_TABLES * DIM), jnp.float32),
    mesh=plsc.VectorSubcoreMesh(
        core_axis_name="c", subcore_axis_name="s",
        num_cores=NC, num_subcores=NS),
    scratch_types=[
        pltpu.VMEM((IDX_ROWS, 128), jnp.int32),
        pltpu.VMEM((IDX_ROWS, 128), jnp.int32),
        pltpu.VMEM((ROWS_PER_CHUNK, ROWP), jnp.float32),
        pltpu.VMEM((ROWS_PER_CHUNK, ROWP), jnp.float32),
        pltpu.VMEM((CHUNK, ROWP), jnp.float32),
        pltpu.VMEM((CHUNK, ROWP), jnp.float32),
        pltpu.SemaphoreType.DMA,
        pltpu.SemaphoreType.DMA,
        pltpu.SemaphoreType.DMA,
        pltpu.SemaphoreType.DMA,
        pltpu.SemaphoreType.DMA,
        pltpu.SemaphoreType.DMA,
    ],
)


@jax.jit
def kernel(indices, tables):
  offs = (jnp.arange(NUM_TABLES, dtype=jnp.int32) * VOCAB)[:, None, None]
  idx = indices.astype(jnp.int32) + offs
  idx = idx.reshape(PAIRS, 2, NW, BLOCKS, CHUNK, L)
  idx = idx.transpose(2, 0, 3, 1, 4, 5).reshape(TOTAL_CHUNKS, ROWS_PER_CHUNK)
  idx = jnp.pad(idx, ((0, 0), (0, IDX_ROWS * 128 - ROWS_PER_CHUNK)))
  idx = idx.reshape(TOTAL_CHUNKS, IDX_ROWS, 128)
  tbl = jnp.pad(tables.reshape(NUM_TABLES * VOCAB, DIM), ((0, 0), (0, DIM)))
  return _emb_kernel(idx, tbl)
